# baseline (speedup 1.0000x reference)
import functools

import jax
import jax.numpy as jnp
from jax import lax
from jax.experimental import pallas as pl
from jax.experimental.pallas import tpu as pltpu
from jax.experimental.pallas import tpu_sc as plsc

SEQ = 4096
BATCH = 4
D_MODEL = 1024

NC = 2
NS = 16
NW = NC * NS
ROWS_PW = SEQ // NW
T = 8
CHUNKS = ROWS_PW // T
LANES = 16
DGRP = D_MODEL // LANES


def _sc_body(x_hbm, t_hbm, o_hbm, xb, tb, x_sem, t_sem, o_sem):
    c = lax.axis_index("c")
    s = lax.axis_index("s")
    wid = s * NC + c
    base = wid * ROWS_PW

    def start_in(slot, chunk):
        t0 = base + chunk * T
        dx = pltpu.async_copy(x_hbm.at[pl.ds(t0, T)], xb.at[slot],
                              x_sem.at[slot])
        dt = pltpu.async_copy(t_hbm.at[pl.ds(t0, T)], tb.at[slot],
                              t_sem.at[slot])
        return dx, dt

    def add_chunk(slot):
        def body(j, carry):
            t = j // DGRP
            d = (j % DGRP) * LANES
            tv = tb[slot, t, pl.ds(d, LANES)]
            for b in range(BATCH):
                xb[slot, t, b, pl.ds(d, LANES)] = (
                    xb[slot, t, b, pl.ds(d, LANES)] + tv)
            return carry
        lax.fori_loop(0, T * DGRP, body, 0, unroll=4)

    in_flight = {}
    out_flight = {}
    in_flight[0] = start_in(0, 0)
    for chunk in range(CHUNKS):
        slot = chunk & 1
        if chunk + 1 < CHUNKS:
            if chunk >= 1:
                out_flight.pop(chunk - 1).wait()
            in_flight[chunk + 1] = start_in(1 - slot, chunk + 1)
        dx, dt = in_flight.pop(chunk)
        dx.wait()
        dt.wait()
        add_chunk(slot)
        out_flight[chunk] = pltpu.async_copy(
            xb.at[slot], o_hbm.at[pl.ds(base + chunk * T, T)],
            o_sem.at[slot])
    out_flight.pop(CHUNKS - 2).wait()
    out_flight.pop(CHUNKS - 1).wait()


@jax.jit
def kernel(x, table):
    s, b, d = x.shape
    mesh = plsc.VectorSubcoreMesh(core_axis_name="c", subcore_axis_name="s")
    f = pl.kernel(
        _sc_body,
        out_type=jax.ShapeDtypeStruct((s, b, d), x.dtype),
        mesh=mesh,
        scratch_types=[
            pltpu.VMEM((2, T, BATCH, D_MODEL), jnp.float32),
            pltpu.VMEM((2, T, D_MODEL), jnp.float32),
            pltpu.SemaphoreType.DMA((2,)),
            pltpu.SemaphoreType.DMA((2,)),
            pltpu.SemaphoreType.DMA((2,)),
        ],
    )
    return f(x, table)


# speedup vs baseline: 1.2408x; 1.2408x over previous
import jax
import jax.numpy as jnp
from jax import lax
from jax.experimental import pallas as pl
from jax.experimental.pallas import tpu as pltpu
from jax.experimental.pallas import tpu_sc as plsc

SEQ = 4096
BATCH = 4
D_MODEL = 1024

NC = 2
NS = 16
NW = NC * NS
T = 8
LANES = 16
DGRP = D_MODEL // LANES

SC_ROWS = 1024
TC_ROWS = SEQ - SC_ROWS
SC_RPW = SC_ROWS // NW
SC_CHUNKS = SC_RPW // T

_BS = 512


def _tc_body(x_ref, t_ref, o_ref):
    o_ref[...] = x_ref[...] + t_ref[...][:, None, :]


def _sc_body(x_hbm, t_hbm, o_hbm, xb, tb, x_sem, t_sem, o_sem):
    c = lax.axis_index("c")
    s = lax.axis_index("s")
    wid = s * NC + c
    base_in = TC_ROWS + wid * SC_RPW
    base_out = wid * SC_RPW

    def start_in(slot, chunk):
        t0 = base_in + chunk * T
        dx = pltpu.async_copy(x_hbm.at[pl.ds(t0, T)], xb.at[slot],
                              x_sem.at[slot])
        dt = pltpu.async_copy(t_hbm.at[pl.ds(t0, T)], tb.at[slot],
                              t_sem.at[slot])
        return dx, dt

    def add_chunk(slot):
        def body(j, carry):
            t = j // DGRP
            d = (j % DGRP) * LANES
            tv = tb[slot, t, pl.ds(d, LANES)]
            for b in range(BATCH):
                xb[slot, t, b, pl.ds(d, LANES)] = (
                    xb[slot, t, b, pl.ds(d, LANES)] + tv)
            return carry
        lax.fori_loop(0, T * DGRP, body, 0, unroll=4)

    in_flight = {0: start_in(0, 0)}
    out_flight = {}
    for chunk in range(SC_CHUNKS):
        slot = chunk & 1
        if chunk + 1 < SC_CHUNKS:
            if chunk >= 1:
                out_flight.pop(chunk - 1).wait()
            in_flight[chunk + 1] = start_in(1 - slot, chunk + 1)
        dx, dt = in_flight.pop(chunk)
        dx.wait()
        dt.wait()
        add_chunk(slot)
        out_flight[chunk] = pltpu.async_copy(
            xb.at[slot], o_hbm.at[pl.ds(base_out + chunk * T, T)],
            o_sem.at[slot])
    for chunk in (SC_CHUNKS - 2, SC_CHUNKS - 1):
        if chunk in out_flight:
            out_flight.pop(chunk).wait()


@jax.jit
def kernel(x, table):
    s, b, d = x.shape

    mesh = plsc.VectorSubcoreMesh(core_axis_name="c", subcore_axis_name="s")
    sc_part = pl.kernel(
        _sc_body,
        out_type=jax.ShapeDtypeStruct((SC_ROWS, b, d), x.dtype),
        mesh=mesh,
        scratch_types=[
            pltpu.VMEM((2, T, BATCH, D_MODEL), jnp.float32),
            pltpu.VMEM((2, T, D_MODEL), jnp.float32),
            pltpu.SemaphoreType.DMA((2,)),
            pltpu.SemaphoreType.DMA((2,)),
            pltpu.SemaphoreType.DMA((2,)),
        ],
    )(x, table)

    tc_full = pl.pallas_call(
        _tc_body,
        grid=(TC_ROWS // _BS,),
        in_specs=[
            pl.BlockSpec((_BS, b, d), lambda i: (i, 0, 0)),
            pl.BlockSpec((_BS, d), lambda i: (i, 0)),
        ],
        out_specs=pl.BlockSpec((_BS, b, d), lambda i: (i, 0, 0)),
        out_shape=jax.ShapeDtypeStruct((s, b, d), x.dtype),
    )(x, table)

    return lax.dynamic_update_slice(tc_full, sc_part, (TC_ROWS, 0, 0))
